# Initial kernel scaffold; baseline (speedup 1.0000x reference)
import jax
import jax.numpy as jnp
from jax.experimental import pallas as pl


def kernel(poi_ids, neighbor_ids, embedding):
    raise NotImplementedError("write your pallas kernel here")


# speedup vs baseline: 4.1564x; 4.1564x over previous
import functools

import jax
import jax.numpy as jnp
from jax import lax
from jax.experimental import pallas as pl
from jax.experimental.pallas import tpu as pltpu
from jax.experimental.pallas import tpu_sc as plsc

L = 16
NC, NS = 2, 16
NW = NC * NS
D = 64
BAG = 8
C = 64
IPC = C * BAG
ROWS_PER_DMA = 128
NDMA = IPC // ROWS_PER_DMA

B_POI = 1024 * 20
B_NB = 1024 * 20 * 8
TOTAL_BAGS = B_POI + B_NB
BAGS_PER_W = TOTAL_BAGS // NW
NCHUNKS = BAGS_PER_W // C


def _sc_body(emb_hbm, ids_hbm, out_hbm, idx_v, rows_v, out_v, sem):
    wid = lax.axis_index("s") * NC + lax.axis_index("c")

    def chunk(c, carry):
        bag_base = wid * BAGS_PER_W + c * C
        pltpu.sync_copy(ids_hbm.at[pl.ds(bag_base * BAG, IPC)], idx_v)
        cps = [
            pltpu.async_copy(emb_hbm.at[idx_v.at[pl.ds(i * ROWS_PER_DMA,
                                                       ROWS_PER_DMA)]],
                             rows_v.at[pl.ds(i * ROWS_PER_DMA, ROWS_PER_DMA), :],
                             sem)
            for i in range(NDMA)
        ]
        recips = []
        for g in range(C // L):
            cnt = jnp.zeros((L,), jnp.float32)
            for j in range(BAG):
                x = idx_v[pl.ds(g * L * BAG + j * L, L)]
                cnt = cnt + jnp.where(x != 0, 1.0, 0.0)
            recips.append(1.0 / jnp.maximum(cnt, 1.0))
        for cp in cps:
            cp.wait()

        for g in range(C // L):
            for l in range(L):
                rec = jnp.broadcast_to(recips[g][l], (L,))
                rbase = g * L * BAG + l
                b = g * L + l
                for k in range(D // L):
                    acc = rows_v[rbase, pl.ds(k * L, L)]
                    for j in range(1, BAG):
                        acc = acc + rows_v[rbase + j * L, pl.ds(k * L, L)]
                    out_v[pl.ds(b * D + k * L, L)] = acc * rec
        pltpu.sync_copy(out_v, out_hbm.at[pl.ds(bag_base * D, C * D)])
        return carry

    lax.fori_loop(0, NCHUNKS, chunk, 0)


@functools.partial(jax.jit, static_argnames=())
def _sc_pool(emb, ids2d):
    kfn = pl.kernel(
        _sc_body,
        out_type=jax.ShapeDtypeStruct((TOTAL_BAGS * D,), jnp.float32),
        mesh=plsc.VectorSubcoreMesh(core_axis_name="c", subcore_axis_name="s"),
        scratch_types=[
            pltpu.VMEM((IPC,), jnp.int32),
            pltpu.VMEM((IPC, D), jnp.float32),
            pltpu.VMEM((C * D,), jnp.float32),
            pltpu.SemaphoreType.DMA,
        ],
        compiler_params=pltpu.CompilerParams(use_tc_tiling_on_sc=False),
    )
    return kfn(emb, ids2d)


def kernel(poi_ids, neighbor_ids, embedding):
    ids = jnp.concatenate(
        [poi_ids.reshape(-1), neighbor_ids.reshape(-1)]).astype(jnp.int32)
    ids_t = ids.reshape(TOTAL_BAGS // L, L, BAG).transpose(0, 2, 1).reshape(-1)
    out = _sc_pool(embedding, ids_t)
    poi = out[: B_POI * D].reshape(1024, 20, D)
    nb = out[B_POI * D:].reshape(1024, 20, BAG, D)
    return (poi, nb)


# speedup vs baseline: 5.5021x; 1.3238x over previous
import functools

import jax
import jax.numpy as jnp
from jax import lax
from jax.experimental import pallas as pl
from jax.experimental.pallas import tpu as pltpu
from jax.experimental.pallas import tpu_sc as plsc

L = 16
NC, NS = 2, 16
NW = NC * NS
D = 64
BAG = 8
C = 32
IPC = C * BAG
ROWS_PER_DMA = 128
NDMA = IPC // ROWS_PER_DMA

B_POI = 1024 * 20
B_NB = 1024 * 20 * 8
TOTAL_BAGS = B_POI + B_NB
BAGS_PER_W = TOTAL_BAGS // NW
NCHUNKS = BAGS_PER_W // C
POI_GCHUNKS = B_POI // C


def _sc_body(emb_hbm, poi_hbm, nb_hbm, opoi_hbm, onb_hbm,
             idx0, idx1, rows0, rows1, out0, out1,
             semg0, semg1, semi0, semi1, semo0, semo1):
    wid = lax.axis_index("s") * NC + lax.axis_index("c")
    idx = (idx0, idx1)
    rows = (rows0, rows1)
    outb = (out0, out1)
    semg = (semg0, semg1)
    semi = (semi0, semi1)
    semo = (semo0, semo1)
    lanes_hi = lax.iota(jnp.int32, L) >= BAG

    def fire_ids(cc, par):
        g = wid * NCHUNKS + cc

        @pl.when(g < POI_GCHUNKS)
        def _():
            pltpu.async_copy(poi_hbm.at[pl.ds(g * IPC, IPC)], idx[par],
                             semi[par])

        @pl.when(g >= POI_GCHUNKS)
        def _():
            pltpu.async_copy(nb_hbm.at[pl.ds((g - POI_GCHUNKS) * IPC, IPC)],
                             idx[par], semi[par])

    def wait_ids(par):
        pltpu.make_async_copy(poi_hbm.at[pl.ds(0, IPC)], idx[par],
                              semi[par]).wait()

    def fire_gathers(par):
        for i in range(NDMA):
            pltpu.async_copy(
                emb_hbm.at[idx[par].at[pl.ds(i * ROWS_PER_DMA,
                                             ROWS_PER_DMA)]],
                rows[par].at[pl.ds(i * ROWS_PER_DMA, ROWS_PER_DMA), :],
                semg[par])

    def wait_gathers(par):
        for i in range(NDMA):
            pltpu.make_async_copy(
                emb_hbm.at[idx[par].at[pl.ds(i * ROWS_PER_DMA,
                                             ROWS_PER_DMA)]],
                rows[par].at[pl.ds(i * ROWS_PER_DMA, ROWS_PER_DMA), :],
                semg[par]).wait()

    def fire_out(cc, par):
        g = wid * NCHUNKS + cc

        @pl.when(g < POI_GCHUNKS)
        def _():
            pltpu.async_copy(outb[par], opoi_hbm.at[pl.ds(g * C * D, C * D)],
                             semo[par])

        @pl.when(g >= POI_GCHUNKS)
        def _():
            pltpu.async_copy(
                outb[par],
                onb_hbm.at[pl.ds((g - POI_GCHUNKS) * C * D, C * D)],
                semo[par])

    def wait_out(par):
        pltpu.make_async_copy(outb[par], opoi_hbm.at[pl.ds(0, C * D)],
                              semo[par]).wait()

    def compute(par):
        for t in range(C // 2):
            v = idx[par][pl.ds(t * L, L)]
            m = jnp.where(v != 0, 1.0, 0.0)
            cs = plsc.cumsum(m)
            c0 = jnp.broadcast_to(cs[BAG - 1], (L,))
            cnts = cs - jnp.where(lanes_hi, c0, 0.0)
            recv = 1.0 / jnp.maximum(cnts, 1.0)
            recs = (jnp.broadcast_to(recv[BAG - 1], (L,)),
                    jnp.broadcast_to(recv[2 * BAG - 1], (L,)))
            for h in range(2):
                b = t * 2 + h
                rbase = b * BAG
                for k in range(D // L):
                    acc = rows[par][rbase, pl.ds(k * L, L)]
                    for j in range(1, BAG):
                        acc = acc + rows[par][rbase + j, pl.ds(k * L, L)]
                    outb[par][pl.ds(b * D + k * L, L)] = acc * recs[h]

    fire_ids(0, 0)
    fire_ids(1, 1)
    wait_ids(0)
    fire_gathers(0)

    def step(c2, carry):
        for par in range(2):
            c = c2 * 2 + par
            wait_gathers(par)

            @pl.when(c + 1 < NCHUNKS)
            def _():
                wait_ids(1 - par)
                fire_gathers(1 - par)

            @pl.when(c >= 2)
            def _():
                wait_out(par)

            compute(par)

            @pl.when(c + 2 < NCHUNKS)
            def _():
                fire_ids(c + 2, par)

            fire_out(c, par)
        return carry

    lax.fori_loop(0, NCHUNKS // 2, step, 0)
    wait_out(0)
    wait_out(1)


@functools.partial(jax.jit, static_argnames=())
def _sc_pool(emb, poi_flat, nb_flat):
    kfn = pl.kernel(
        _sc_body,
        out_type=(jax.ShapeDtypeStruct((B_POI * D,), jnp.float32),
                  jax.ShapeDtypeStruct((B_NB * D,), jnp.float32)),
        mesh=plsc.VectorSubcoreMesh(core_axis_name="c", subcore_axis_name="s"),
        scratch_types=[
            pltpu.VMEM((IPC,), jnp.int32),
            pltpu.VMEM((IPC,), jnp.int32),
            pltpu.VMEM((IPC, D), jnp.float32),
            pltpu.VMEM((IPC, D), jnp.float32),
            pltpu.VMEM((C * D,), jnp.float32),
            pltpu.VMEM((C * D,), jnp.float32),
            pltpu.SemaphoreType.DMA,
            pltpu.SemaphoreType.DMA,
            pltpu.SemaphoreType.DMA,
            pltpu.SemaphoreType.DMA,
            pltpu.SemaphoreType.DMA,
            pltpu.SemaphoreType.DMA,
        ],
        compiler_params=pltpu.CompilerParams(use_tc_tiling_on_sc=False,
                                             needs_layout_passes=False),
    )
    return kfn(emb, poi_flat, nb_flat)


def kernel(poi_ids, neighbor_ids, embedding):
    poi_flat = poi_ids.reshape(-1).astype(jnp.int32)
    nb_flat = neighbor_ids.reshape(-1).astype(jnp.int32)
    opoi, onb = _sc_pool(embedding, poi_flat, nb_flat)
    return (opoi.reshape(1024, 20, D), onb.reshape(1024, 20, BAG, D))
